# baseline (speedup 1.0000x reference)
import jax
import jax.numpy as jnp
from jax import lax
from jax.experimental import pallas as pl
from jax.experimental.pallas import tpu as pltpu
from jax.experimental.pallas import tpu_sc as plsc

VOCAB = 1_000_000
D = 64
BATCH = 4096
SEQ = 50
PW = 128

_info = plsc.get_sparse_core_info()
NC = _info.num_cores
NS = _info.num_subcores
NW = NC * NS
CH = BATCH // NW
NB = 2
L = 16


def _make_lookup():
  mesh = plsc.VectorSubcoreMesh(core_axis_name="c", subcore_axis_name="s")

  @pl.kernel(
      out_type=jax.ShapeDtypeStruct((SEQ, D, BATCH), jnp.float32),
      mesh=mesh,
      compiler_params=pltpu.CompilerParams(needs_layout_passes=False),
      scratch_types=(
          [pltpu.VMEM((SEQ, CH), jnp.int32),
           pltpu.VMEM((SEQ, CH), jnp.int32)]
          + [pltpu.VMEM((CH, PW), jnp.float32) for _ in range(NB)]
          + [pltpu.VMEM((D, CH), jnp.float32) for _ in range(NB)]
          + [pltpu.SemaphoreType.DMA for _ in range(2 * NB)]
      ),
  )
  def lookup(tp_hbm, xt_hbm, out_hbm, idx_v, par_v, *bufs_sems):
    gbufs = bufs_sems[:NB]
    tbufs = bufs_sems[NB:2 * NB]
    sg = bufs_sems[2 * NB:3 * NB]
    sw = bufs_sems[3 * NB:4 * NB]
    wid = lax.axis_index("s") * NC + lax.axis_index("c")
    b0 = wid * CH
    pltpu.sync_copy(xt_hbm.at[:, pl.ds(b0, CH)], idx_v)

    @pl.loop(0, SEQ)
    def _prep(s):
      for j in range(CH // L):
        par_v[s, pl.ds(j * L, L)] = lax.shift_right_logical(
            idx_v[s, pl.ds(j * L, L)], 1)

    def out_slice(s):
      return out_hbm.at[s, :, pl.ds(b0, CH)]

    def start_gather(s, b):
      pltpu.async_copy(tp_hbm.at[par_v.at[s]], gbufs[b], sg[b])

    iota = lax.iota(jnp.int32, L)

    for b in range(NB):
      start_gather(b, b)

    @pl.loop(0, SEQ, step=NB)
    def _chunks(s0):
      for b in range(NB):
        s = s0 + b
        pltpu.make_async_copy(tp_hbm.at[par_v.at[s]], gbufs[b], sg[b]).wait()

        @pl.when(s >= NB)
        def _():
          pltpu.make_async_copy(tbufs[b], out_slice(s - NB), sw[b]).wait()

        for j in range(CH // L):
          rows = iota + (j * L)
          off = (idx_v[s, pl.ds(j * L, L)] & 1) * D
          for d in range(D):
            tbufs[b][d, pl.ds(j * L, L)] = plsc.load_gather(
                gbufs[b], [rows, off + d])

        pltpu.async_copy(tbufs[b], out_slice(s), sw[b])

        @pl.when(s + NB < SEQ)
        def _():
          start_gather(s + NB, b)

    for b in range(NB):
      s = SEQ - NB + b
      pltpu.make_async_copy(tbufs[b], out_slice(s), sw[b]).wait()

  return lookup


_lookup = _make_lookup()


@jax.jit
def kernel(x, table):
  tp = table.reshape(VOCAB // 2, PW)
  o2 = _lookup(tp, x.T.astype(jnp.int32))
  return o2.transpose(2, 0, 1)


# speedup vs baseline: 1.8714x; 1.8714x over previous
import jax
import jax.numpy as jnp
from jax import lax
from jax.experimental import pallas as pl
from jax.experimental.pallas import tpu as pltpu
from jax.experimental.pallas import tpu_sc as plsc

VOCAB = 1_000_000
D = 64
BATCH = 4096
SEQ = 50

_info = plsc.get_sparse_core_info()
NC = _info.num_cores
NS = _info.num_subcores
NW = NC * NS
CH = BATCH // NW
NB = 2


def _make_lookup():
  mesh = plsc.VectorSubcoreMesh(core_axis_name="c", subcore_axis_name="s")

  @pl.kernel(
      out_type=jax.ShapeDtypeStruct((BATCH, SEQ, D), jnp.float32),
      mesh=mesh,
      scratch_types=(
          [pltpu.VMEM((SEQ, CH), jnp.int32)]
          + [pltpu.VMEM((CH, D), jnp.float32) for _ in range(NB)]
          + [pltpu.SemaphoreType.DMA for _ in range(2 * NB)]
      ),
  )
  def lookup(t_hbm, xt_hbm, out_hbm, idx_v, *bufs_sems):
    gbufs = bufs_sems[:NB]
    sg = bufs_sems[NB:2 * NB]
    sw = bufs_sems[2 * NB:3 * NB]
    wid = lax.axis_index("s") * NC + lax.axis_index("c")
    b0 = wid * CH
    pltpu.sync_copy(xt_hbm.at[:, pl.ds(b0, CH)], idx_v)

    def rowdma(s, b):
      @pl.loop(0, CH // 16)
      def _rows(g):
        vv = idx_v[s, pl.ds(g * 16, 16)]
        for l in range(16):
          pltpu.async_copy(t_hbm.at[vv[l]], gbufs[b].at[g * 16 + l], sg[b])

    def out_slice(s):
      return out_hbm.at[pl.ds(b0, CH), s, :]

    rowdma(0, 0)

    @pl.loop(0, SEQ, step=NB)
    def _chunks(s0):
      for b in range(NB):
        s = s0 + b
        b2 = (b + 1) % NB

        @pl.when(s + 1 < SEQ)
        def _():
          @pl.when(s >= 1)
          def _():
            pltpu.make_async_copy(gbufs[b2], out_slice(s - 1), sw[b2]).wait()
          rowdma(s + 1, b2)

        pltpu.make_async_copy(t_hbm.at[pl.ds(0, CH), :], gbufs[b], sg[b]).wait()
        pltpu.async_copy(gbufs[b], out_slice(s), sw[b])

    for b in range(NB):
      s = SEQ - NB + b
      pltpu.make_async_copy(gbufs[b], out_slice(s), sw[b]).wait()

  return lookup


_lookup = _make_lookup()


@jax.jit
def kernel(x, table):
  return _lookup(table, x.T.astype(jnp.int32))


# speedup vs baseline: 2.4448x; 1.3064x over previous
import jax
import jax.numpy as jnp
from jax import lax
from jax.experimental import pallas as pl
from jax.experimental.pallas import tpu as pltpu
from jax.experimental.pallas import tpu_sc as plsc

VOCAB = 1_000_000
D = 64
BATCH = 4096
SEQ = 50

_info = plsc.get_sparse_core_info()
NC = _info.num_cores
NS = _info.num_subcores
NW = NC * NS
CH = BATCH // NW
NB = 2


def _make_lookup():
  mesh = plsc.VectorSubcoreMesh(core_axis_name="c", subcore_axis_name="s")

  @pl.kernel(
      out_type=jax.ShapeDtypeStruct((BATCH, SEQ, D), jnp.float32),
      mesh=mesh,
      scratch_types=(
          [pltpu.VMEM((SEQ, CH), jnp.int32)]
          + [pltpu.VMEM((CH, D), jnp.float32) for _ in range(NB)]
          + [pltpu.SemaphoreType.DMA for _ in range(2 * NB)]
      ),
  )
  def lookup(t_hbm, xt_hbm, out_hbm, idx_v, *bufs_sems):
    gbufs = bufs_sems[:NB]
    sg = bufs_sems[NB:2 * NB]
    sw = bufs_sems[2 * NB:3 * NB]
    wid = lax.axis_index("s") * NC + lax.axis_index("c")
    b0 = wid * CH
    pltpu.sync_copy(xt_hbm.at[:, pl.ds(b0, CH)], idx_v)

    def rowdma(s, b):
      @pl.loop(0, CH // 16)
      def _rows(g):
        vv = idx_v[s, pl.ds(g * 16, 16)]
        hh = lax.div(vv, VOCAB // 4)
        ll = lax.rem(vv, VOCAB // 4)
        for l in range(16):
          pltpu.async_copy(
              t_hbm.at[hh[l], ll[l]], gbufs[b].at[g * 16 + l], sg[b])

    def out_slice(s):
      return out_hbm.at[pl.ds(b0, CH), s, :]

    rowdma(0, 0)

    @pl.loop(0, SEQ, step=NB)
    def _chunks(s0):
      for b in range(NB):
        s = s0 + b
        b2 = (b + 1) % NB

        @pl.when(s + 1 < SEQ)
        def _():
          @pl.when(s >= 1)
          def _():
            pltpu.make_async_copy(gbufs[b2], out_slice(s - 1), sw[b2]).wait()
          rowdma(s + 1, b2)

        pltpu.make_async_copy(
            t_hbm.at[0, pl.ds(0, CH), :], gbufs[b], sg[b]).wait()
        pltpu.async_copy(gbufs[b], out_slice(s), sw[b])

    for b in range(NB):
      s = SEQ - NB + b
      pltpu.make_async_copy(gbufs[b], out_slice(s), sw[b]).wait()

  return lookup


_lookup = _make_lookup()


@jax.jit
def kernel(x, table):
  t4 = table.reshape(4, VOCAB // 4, D)
  return _lookup(t4, x.T.astype(jnp.int32))


# speedup vs baseline: 2.6519x; 1.0847x over previous
import jax
import jax.numpy as jnp
from jax import lax
from jax.experimental import pallas as pl
from jax.experimental.pallas import tpu as pltpu
from jax.experimental.pallas import tpu_sc as plsc

VOCAB = 1_000_000
D = 64
BATCH = 4096
SEQ = 50

_info = plsc.get_sparse_core_info()
NC = _info.num_cores
NS = _info.num_subcores
NW = NC * NS
CH = BATCH // NW
NB = 2


def _make_lookup():
  mesh = plsc.VectorSubcoreMesh(core_axis_name="c", subcore_axis_name="s")

  @pl.kernel(
      out_type=jax.ShapeDtypeStruct((NW, CH, SEQ, D), jnp.float32),
      mesh=mesh,
      scratch_types=(
          [pltpu.VMEM((SEQ, CH), jnp.int32)]
          + [pltpu.VMEM((CH, D), jnp.float32) for _ in range(NB)]
          + [pltpu.SemaphoreType.DMA for _ in range(2 * NB)]
      ),
  )
  def lookup(t_hbm, xt_hbm, out_hbm, idx_v, *bufs_sems):
    gbufs = bufs_sems[:NB]
    sg = bufs_sems[NB:2 * NB]
    sw = bufs_sems[2 * NB:3 * NB]
    wid = lax.axis_index("s") * NC + lax.axis_index("c")
    b0 = wid * CH
    pltpu.sync_copy(xt_hbm.at[:, pl.ds(b0, CH)], idx_v)

    def rowdma(s, b):
      @pl.loop(0, CH // 16)
      def _rows(g):
        vv = idx_v[s, pl.ds(g * 16, 16)]
        hh = lax.div(vv, VOCAB // 4)
        ll = lax.rem(vv, VOCAB // 4)
        for l in range(16):
          pltpu.async_copy(
              t_hbm.at[hh[l], ll[l]], gbufs[b].at[g * 16 + l], sg[b])

    def out_slice(s):
      return out_hbm.at[wid, :, s, :]

    rowdma(0, 0)

    @pl.loop(0, SEQ, step=NB)
    def _chunks(s0):
      for b in range(NB):
        s = s0 + b
        b2 = (b + 1) % NB

        @pl.when(s + 1 < SEQ)
        def _():
          @pl.when(s >= 1)
          def _():
            pltpu.make_async_copy(gbufs[b2], out_slice(s - 1), sw[b2]).wait()
          rowdma(s + 1, b2)

        pltpu.make_async_copy(
            t_hbm.at[0, pl.ds(0, CH), :], gbufs[b], sg[b]).wait()
        pltpu.async_copy(gbufs[b], out_slice(s), sw[b])

    for b in range(NB):
      s = SEQ - NB + b
      pltpu.make_async_copy(gbufs[b], out_slice(s), sw[b]).wait()

  return lookup


_lookup = _make_lookup()


@jax.jit
def kernel(x, table):
  t4 = table.reshape(4, VOCAB // 4, D)
  o4 = _lookup(t4, x.T.astype(jnp.int32))
  return o4.reshape(BATCH, SEQ, D)
